# Initial kernel scaffold; baseline (speedup 1.0000x reference)
import jax
import jax.numpy as jnp
from jax.experimental import pallas as pl


def kernel(u_pos, v_pos, v_neg, batch_size, u_weight, v_weight):
    raise NotImplementedError("write your pallas kernel here")


# speedup vs baseline: 2.8633x; 2.8633x over previous
import functools

import jax
import jax.numpy as jnp
from jax import lax
from jax.experimental import pallas as pl
from jax.experimental.pallas import tpu as pltpu
from jax.experimental.pallas import tpu_sc as plsc

_NC = 2
_NS = 16
_NW = _NC * _NS
_L = 16


@functools.lru_cache(maxsize=None)
def _make_sc(B, D, NNEG, interpret=False):
    CHUNK = B // _NW
    S = 64
    NSUB = CHUNK // S
    SN = S * NNEG
    KD = D // _L
    mesh = plsc.VectorSubcoreMesh(core_axis_name="c", subcore_axis_name="s",
                                  num_cores=_NC, num_subcores=_NS)

    @functools.partial(
        pl.kernel, mesh=mesh, interpret=interpret,
        compiler_params=pltpu.CompilerParams(use_tc_tiling_on_sc=False),
        out_type=(jax.ShapeDtypeStruct((B, _L), jnp.float32),
                  jax.ShapeDtypeStruct((B, _L), jnp.float32)),
        scratch_types=[
            pltpu.VMEM((S,), jnp.int32),
            pltpu.VMEM((S,), jnp.int32),
            pltpu.VMEM((SN,), jnp.int32),
            pltpu.VMEM((S, D), jnp.float32),
            pltpu.VMEM((S, D), jnp.float32),
            pltpu.VMEM((SN, D), jnp.float32),
            pltpu.VMEM((S, _L), jnp.float32),
            pltpu.VMEM((S, _L), jnp.float32),
            pltpu.SemaphoreType.DMA,
        ],
    )
    def sc_fn(upos, vpos, vnegf, uw, vw, spart, npart,
              uidx, vidx, nidx, urows, vrows, nrows, sbuf, nbuf, sem):
        wid = lax.axis_index("s") * _NC + lax.axis_index("c")
        base = wid * CHUNK

        def sub(j, carry):
            gb = pl.multiple_of(base + j * S, S)
            pltpu.sync_copy(upos.at[pl.ds(gb, S)], uidx)
            pltpu.sync_copy(vpos.at[pl.ds(gb, S)], vidx)
            pltpu.sync_copy(vnegf.at[pl.ds(gb * NNEG, SN)], nidx)
            cps = [pltpu.async_copy(uw.at[uidx], urows, sem),
                   pltpu.async_copy(vw.at[vidx], vrows, sem)]
            for c in range(SN // 128):
                cps.append(pltpu.async_copy(
                    vw.at[nidx.at[pl.ds(c * 128, 128)]],
                    nrows.at[pl.ds(c * 128, 128)], sem))
            for cp in cps:
                cp.wait()

            def body(b, acc_carry):
                su = [urows[b, pl.ds(k * _L, _L)] for k in range(KD)]
                sv = [vrows[b, pl.ds(k * _L, _L)] for k in range(KD)]
                ps = su[0] * sv[0]
                for k in range(1, KD):
                    ps = ps + su[k] * sv[k]
                sbuf[b, :] = ps
                acc = None
                for n in range(NNEG):
                    r = b * NNEG + n
                    for k in range(KD):
                        t = nrows[r, pl.ds(k * _L, _L)] * su[k]
                        acc = t if acc is None else acc + t
                nbuf[b, :] = acc
                return acc_carry

            lax.fori_loop(0, S, body, 0)
            pltpu.sync_copy(sbuf, spart.at[pl.ds(gb, S)])
            pltpu.sync_copy(nbuf, npart.at[pl.ds(gb, S)])
            return carry

        lax.fori_loop(0, NSUB, sub, 0)

    return sc_fn


def _tc_finish(spart, npart, interpret=False):
    def body(s_ref, n_ref, o_ref):
        s = jnp.sum(s_ref[...], axis=1)
        ns = jnp.sum(n_ref[...], axis=1)
        tot = jnp.sum(jax.nn.log_sigmoid(s) + jax.nn.log_sigmoid(-ns))
        o_ref[0, 0] = tot
    out = pl.pallas_call(
        body,
        out_shape=jax.ShapeDtypeStruct((1, 1), jnp.float32),
        out_specs=pl.BlockSpec(memory_space=pltpu.SMEM),
        interpret=interpret,
    )(spart, npart)
    return out[0, 0]


def kernel(u_pos, v_pos, v_neg, batch_size, u_weight, v_weight):
    B = u_pos.shape[0]
    D = u_weight.shape[1]
    NNEG = v_neg.shape[1]
    up = u_pos.astype(jnp.int32)
    vp = v_pos.astype(jnp.int32)
    vn = v_neg.astype(jnp.int32).reshape(-1)
    spart, npart = _make_sc(B, D, NNEG)(up, vp, vn, u_weight, v_weight)
    tot = _tc_finish(spart, npart)
    return -tot / batch_size
